# baseline (speedup 1.0000x reference)
import functools

import jax
import jax.numpy as jnp
from jax import lax
from jax.experimental import pallas as pl
from jax.experimental.pallas import tpu as pltpu
from jax.experimental.pallas import tpu_sc as plsc

N = 10000
E = 320000
D = 128
R = 8

NC = 2
NS = 16
NW = NC * NS
EPW = E // NW
CHUNK = 80
EB = 2000
NBLK = EPW // EB
CPB = EB // CHUNK
N_PAD = 10240
ROWS_PER_TILE = N_PAD // NS


def _dense_body(x_ref, w_ref, gw_ref, out_ref):
    x = x_ref[...]
    gates = jax.nn.sigmoid(
        jnp.dot(x, gw_ref[...], preferred_element_type=jnp.float32))
    for r in range(R):
        t = jnp.dot(x, w_ref[r], preferred_element_type=jnp.float32)
        out_ref[r] = t * gates[:, r:r + 1]


def _dense_table(x, weight, gw2):
    BN = 1000
    NB = N // BN
    return pl.pallas_call(
        _dense_body,
        grid=(NB,),
        in_specs=[
            pl.BlockSpec((BN, D), lambda nb: (nb, 0)),
            pl.BlockSpec((R, D, D), lambda nb: (0, 0, 0)),
            pl.BlockSpec((D, R), lambda nb: (0, 0)),
        ],
        out_specs=pl.BlockSpec((R, BN, D), lambda nb: (0, nb, 0)),
        out_shape=jax.ShapeDtypeStruct((R, N, D), jnp.float32),
    )(x, weight, gw2)


def _prep_body(ei_ref, rel_ref, nrm_ref, gidx_ref, dsto_ref, nrmo_ref):
    gidx_ref[...] = rel_ref[...] * N + ei_ref[0]
    dsto_ref[...] = ei_ref[1]
    nrmo_ref[...] = nrm_ref[...]


def _prep(edge_index, rel, nrm2d):
    return pl.pallas_call(
        _prep_body,
        out_shape=[jax.ShapeDtypeStruct((E,), jnp.int32),
                   jax.ShapeDtypeStruct((E,), jnp.int32),
                   jax.ShapeDtypeStruct((E // 128, 128), jnp.float32)],
    )(edge_index, rel, nrm2d)


def _edge_body(table_hbm, gidx_hbm, dst_hbm, norm_hbm, out_hbm,
               h_sh, gidx_v, dst1d_v, dst_v2, norm_v, rows0, rows1, rows2,
               gsem0, gsem1, gsem2, ssem0, ssem1, ssem2):
    cid = lax.axis_index("c")
    sid = lax.axis_index("s")
    wid = sid * NC + cid
    ebase = wid * EPW

    def zero_row(i, _):
        for j in range(D // 16):
            rows0[i, pl.ds(j * 16, 16)] = jnp.zeros((16,), jnp.float32)
        return 0
    lax.fori_loop(0, CHUNK, zero_row, 0)
    for k in range(ROWS_PER_TILE // CHUNK):
        pltpu.sync_copy(rows0, h_sh.at[pl.ds(sid * ROWS_PER_TILE + k * CHUNK, CHUNK)])
    plsc.subcore_barrier()

    bufs = (rows0, rows1, rows2)
    gsems = (gsem0, gsem1, gsem2)
    ssems = (ssem0, ssem1, ssem2)

    def start_gather(c, b):
        pltpu.async_copy(table_hbm.at[gidx_v.at[pl.ds(c * CHUNK, CHUNK)]],
                         bufs[b], gsems[b])

    def wait_gather(b):
        pltpu.make_async_copy(table_hbm.at[gidx_v.at[pl.ds(0, CHUNK)]],
                              bufs[b], gsems[b]).wait()

    def start_scatter(c, b):
        pltpu.async_copy(bufs[b], h_sh.at[dst_v2.at[c]], ssems[b], add=True)

    def wait_scatter(b):
        pltpu.make_async_copy(bufs[b], h_sh.at[dst_v2.at[0]], ssems[b]).wait()

    def scale(c, b):
        buf = bufs[b]

        @plsc.parallel_loop(0, CHUNK // 16)
        def scale_group(t):
            nv16 = norm_v[pl.ds(c * CHUNK + t * 16, 16)]
            for l in range(16):
                e = t * 16 + l
                nv = nv16[l]
                for dg in range(D // 16):
                    sl = pl.ds(dg * 16, 16)
                    buf[e, sl] = buf[e, sl] * nv

    def block_body(blk, _):
        pltpu.sync_copy(gidx_hbm.at[pl.ds(ebase + blk * EB, EB)], gidx_v)
        pltpu.sync_copy(norm_hbm.at[pl.ds(ebase + blk * EB, EB)], norm_v)
        pltpu.sync_copy(dst_hbm.at[pl.ds(ebase + blk * EB, EB)], dst1d_v)
        def restage(c, _):
            for j in range(CHUNK // 16):
                dst_v2[c, pl.ds(j * 16, 16)] = dst1d_v[pl.ds(c * CHUNK + j * 16, 16)]
            return 0
        lax.fori_loop(0, CPB, restage, 0)

        start_gather(0, 0)
        start_gather(1, 1)
        wait_gather(0)
        scale(0, 0)
        start_scatter(0, 0)
        start_gather(2, 2)
        wait_gather(1)
        scale(1, 1)
        start_scatter(1, 1)
        wait_scatter(0)
        start_gather(3, 0)
        wait_gather(2)
        scale(2, 2)
        start_scatter(2, 2)
        wait_scatter(1)
        start_gather(4, 1)

        def pipe_body(i, _):
            for j in range(3):
                c = 3 * i + j
                b = j
                bn = (j + 2) % 3
                wait_gather(b)
                scale(c, b)
                start_scatter(c, b)

                @pl.when(c + 2 < CPB)
                def _():
                    wait_scatter(bn)
                    start_gather(c + 2, bn)
            return 0
        lax.fori_loop(1, (CPB - 1) // 3, pipe_body, 0)
        wait_gather(0)
        scale(CPB - 1, 0)
        start_scatter(CPB - 1, 0)
        wait_scatter(1)
        wait_scatter(2)
        wait_scatter(0)
        return 0
    lax.fori_loop(0, NBLK, block_body, 0)
    plsc.subcore_barrier()

    for k in range(ROWS_PER_TILE // CHUNK):
        row0 = sid * ROWS_PER_TILE + k * CHUNK
        pltpu.sync_copy(h_sh.at[pl.ds(row0, CHUNK)], rows0)
        pltpu.sync_copy(rows0, out_hbm.at[cid, pl.ds(row0, CHUNK)])


_edge_kernel = functools.partial(
    pl.kernel,
    mesh=plsc.VectorSubcoreMesh(core_axis_name="c", subcore_axis_name="s"),
    out_type=jax.ShapeDtypeStruct((NC, N_PAD, D), jnp.float32),
    scratch_types=[
        pltpu.VMEM_SHARED((N_PAD, D), jnp.float32),
        pltpu.VMEM((EB,), jnp.int32),
        pltpu.VMEM((EB,), jnp.int32),
        pltpu.VMEM((CPB, CHUNK), jnp.int32),
        pltpu.VMEM((EB,), jnp.float32),
        pltpu.VMEM((CHUNK, D), jnp.float32),
        pltpu.VMEM((CHUNK, D), jnp.float32),
        pltpu.VMEM((CHUNK, D), jnp.float32),
        pltpu.SemaphoreType.DMA,
        pltpu.SemaphoreType.DMA,
        pltpu.SemaphoreType.DMA,
        pltpu.SemaphoreType.DMA,
        pltpu.SemaphoreType.DMA,
        pltpu.SemaphoreType.DMA,
    ],
)(_edge_body)


def _combine_body(p_ref, out_ref):
    out_ref[...] = jnp.maximum(p_ref[0] + p_ref[1], 0.0)


def _combine(partials):
    BN = 1000
    NB = N // BN
    return pl.pallas_call(
        _combine_body,
        grid=(NB,),
        in_specs=[pl.BlockSpec((NC, BN, D), lambda i: (0, i, 0))],
        out_specs=pl.BlockSpec((BN, D), lambda i: (i, 0)),
        out_shape=jax.ShapeDtypeStruct((N, D), jnp.float32),
    )(partials)


def kernel(x, edge_index, rel_type, norm, weight, gate_weight):
    table = _dense_table(x, weight, gate_weight[:, :, 0].T)
    table2d = table.reshape(R * N, D)
    gidx, dst, nrm = _prep(edge_index, rel_type, norm.reshape(E // 128, 128))
    partials = _edge_kernel(table2d, gidx, dst, nrm.reshape(E))
    return _combine(partials)


# speedup vs baseline: 1.1907x; 1.1907x over previous
import functools

import jax
import jax.numpy as jnp
from jax import lax
from jax.experimental import pallas as pl
from jax.experimental.pallas import tpu as pltpu
from jax.experimental.pallas import tpu_sc as plsc

N = 10000
E = 320000
D = 128
R = 8

NC = 2
NS = 16
NW = NC * NS
EPW = E // NW
CHUNK = 80
EB = 2000
NBLK = EPW // EB
CPB = EB // CHUNK
N_PAD = 10240
ROWS_PER_TILE = N_PAD // NS


def _dense_body(x_ref, w_ref, gw_ref, out_ref):
    x = x_ref[...]
    gates = jax.nn.sigmoid(
        jnp.dot(x, gw_ref[...], preferred_element_type=jnp.float32))
    for r in range(R):
        t = jnp.dot(x, w_ref[r], preferred_element_type=jnp.float32)
        out_ref[r] = t * gates[:, r:r + 1]


def _dense_table(x, weight, gw2):
    BN = 1000
    NB = N // BN
    return pl.pallas_call(
        _dense_body,
        grid=(NB,),
        in_specs=[
            pl.BlockSpec((BN, D), lambda nb: (nb, 0)),
            pl.BlockSpec((R, D, D), lambda nb: (0, 0, 0)),
            pl.BlockSpec((D, R), lambda nb: (0, 0)),
        ],
        out_specs=pl.BlockSpec((R, BN, D), lambda nb: (0, nb, 0)),
        out_shape=jax.ShapeDtypeStruct((R, N, D), jnp.float32),
    )(x, weight, gw2)


def _prep_body(ei_ref, rel_ref, nrm_ref, gidx_ref, dsto_ref, nrmo_ref):
    gidx_ref[...] = rel_ref[...] * N + ei_ref[0]
    dsto_ref[...] = ei_ref[1]
    nrmo_ref[...] = nrm_ref[...]


def _prep(edge_index, rel, nrm2d):
    return pl.pallas_call(
        _prep_body,
        out_shape=[jax.ShapeDtypeStruct((E,), jnp.int32),
                   jax.ShapeDtypeStruct((E,), jnp.int32),
                   jax.ShapeDtypeStruct((E // 128, 128), jnp.float32)],
    )(edge_index, rel, nrm2d)


def _edge_body(table_hbm, gidx_hbm, dst_hbm4, norm_hbm, out_hbm,
               h_sh, gidx_v, dst_v2, norm_v, rows0, rows1, rows2, rows3,
               gsem0, gsem1, gsem2, gsem3, ssem0, ssem1, ssem2, ssem3):
    cid = lax.axis_index("c")
    sid = lax.axis_index("s")
    wid = sid * NC + cid
    ebase = wid * EPW

    def zero_row(i, _):
        for j in range(D // 16):
            rows0[i, pl.ds(j * 16, 16)] = jnp.zeros((16,), jnp.float32)
        return 0
    lax.fori_loop(0, CHUNK, zero_row, 0)
    for k in range(ROWS_PER_TILE // CHUNK):
        pltpu.sync_copy(rows0, h_sh.at[pl.ds(sid * ROWS_PER_TILE + k * CHUNK, CHUNK)])
    plsc.subcore_barrier()

    bufs = (rows0, rows1, rows2, rows3)
    gsems = (gsem0, gsem1, gsem2, gsem3)
    ssems = (ssem0, ssem1, ssem2, ssem3)

    def start_gather(c, b):
        pltpu.async_copy(table_hbm.at[gidx_v.at[pl.ds(c * CHUNK, CHUNK)]],
                         bufs[b], gsems[b])

    def wait_gather(b):
        pltpu.make_async_copy(table_hbm.at[gidx_v.at[pl.ds(0, CHUNK)]],
                              bufs[b], gsems[b]).wait()

    def start_scatter(c, b):
        pltpu.async_copy(bufs[b], h_sh.at[dst_v2.at[c]], ssems[b], add=True)

    def wait_scatter(b):
        pltpu.make_async_copy(bufs[b], h_sh.at[dst_v2.at[0]], ssems[b]).wait()

    def scale(c, b):
        buf = bufs[b]

        def scale_group(t, _):
            nv16 = norm_v[pl.ds(c * CHUNK + t * 16, 16)]
            for l in range(16):
                e = t * 16 + l
                nv = nv16[l]
                for dg in range(D // 16):
                    sl = pl.ds(dg * 16, 16)
                    buf[e, sl] = buf[e, sl] * nv
            return 0
        lax.fori_loop(0, CHUNK // 16, scale_group, 0)

    def block_body(blk, _):
        pltpu.sync_copy(gidx_hbm.at[pl.ds(ebase + blk * EB, EB)], gidx_v)
        pltpu.sync_copy(norm_hbm.at[pl.ds(ebase + blk * EB, EB)], norm_v)
        pltpu.sync_copy(dst_hbm4.at[wid, blk], dst_v2)

        start_gather(0, 0)
        start_gather(1, 1)
        start_gather(2, 2)
        wait_gather(0)
        scale(0, 0)
        start_scatter(0, 0)
        start_gather(3, 3)
        wait_gather(1)
        scale(1, 1)
        start_scatter(1, 1)
        wait_scatter(0)
        start_gather(4, 0)
        wait_gather(2)
        scale(2, 2)
        start_scatter(2, 2)
        wait_scatter(1)
        start_gather(5, 1)
        wait_gather(3)
        scale(3, 3)
        start_scatter(3, 3)
        wait_scatter(2)
        start_gather(6, 2)

        def pipe_body(i, _):
            for j in range(4):
                c = 4 * i + j
                b = j
                bn = (j + 3) % 4
                wait_gather(b)
                scale(c, b)
                start_scatter(c, b)

                @pl.when(c + 3 < CPB)
                def _():
                    wait_scatter(bn)
                    start_gather(c + 3, bn)
            return 0
        lax.fori_loop(1, (CPB - 1) // 4, pipe_body, 0)
        wait_gather(0)
        scale(CPB - 1, 0)
        start_scatter(CPB - 1, 0)
        wait_scatter(1)
        wait_scatter(2)
        wait_scatter(3)
        wait_scatter(0)
        return 0
    lax.fori_loop(0, NBLK, block_body, 0)
    plsc.subcore_barrier()

    for k in range(ROWS_PER_TILE // CHUNK):
        row0 = sid * ROWS_PER_TILE + k * CHUNK
        pltpu.sync_copy(h_sh.at[pl.ds(row0, CHUNK)], rows0)
        pltpu.sync_copy(rows0, out_hbm.at[cid, pl.ds(row0, CHUNK)])


_edge_kernel = functools.partial(
    pl.kernel,
    mesh=plsc.VectorSubcoreMesh(core_axis_name="c", subcore_axis_name="s"),
    out_type=jax.ShapeDtypeStruct((NC, N_PAD, D), jnp.float32),
    scratch_types=[
        pltpu.VMEM_SHARED((N_PAD, D), jnp.float32),
        pltpu.VMEM((EB,), jnp.int32),
        pltpu.VMEM((CPB, CHUNK), jnp.int32),
        pltpu.VMEM((EB,), jnp.float32),
        pltpu.VMEM((CHUNK, D), jnp.float32),
        pltpu.VMEM((CHUNK, D), jnp.float32),
        pltpu.VMEM((CHUNK, D), jnp.float32),
        pltpu.VMEM((CHUNK, D), jnp.float32),
        pltpu.SemaphoreType.DMA,
        pltpu.SemaphoreType.DMA,
        pltpu.SemaphoreType.DMA,
        pltpu.SemaphoreType.DMA,
        pltpu.SemaphoreType.DMA,
        pltpu.SemaphoreType.DMA,
        pltpu.SemaphoreType.DMA,
        pltpu.SemaphoreType.DMA,
    ],
)(_edge_body)


def _combine_body(p_ref, out_ref):
    out_ref[...] = jnp.maximum(p_ref[0] + p_ref[1], 0.0)


def _combine(partials):
    BN = 1000
    NB = N // BN
    return pl.pallas_call(
        _combine_body,
        grid=(NB,),
        in_specs=[pl.BlockSpec((NC, BN, D), lambda i: (0, i, 0))],
        out_specs=pl.BlockSpec((BN, D), lambda i: (i, 0)),
        out_shape=jax.ShapeDtypeStruct((N, D), jnp.float32),
    )(partials)


def kernel(x, edge_index, rel_type, norm, weight, gate_weight):
    table = _dense_table(x, weight, gate_weight[:, :, 0].T)
    table2d = table.reshape(R * N, D)
    gidx, dst, nrm = _prep(edge_index, rel_type, norm.reshape(E // 128, 128))
    partials = _edge_kernel(table2d, gidx, dst.reshape(NW, NBLK, CPB, CHUNK), nrm.reshape(E))
    return _combine(partials)
